# baseline (speedup 1.0000x reference)
import functools

import jax
import jax.numpy as jnp
from jax.experimental import pallas as pl
from jax.experimental.pallas import tpu as pltpu
from jax.experimental.pallas import tpu_sc as plsc

B, N, K = 8, 2048, 32
R1 = 256
R4 = 1024
TOT = B * N * K
EPS = 1e-5


CH1 = 128
NCH = N // CH1
J1 = 10
NCAND = NCH * J1


def _k1_body(xyz_ref, xyzT_ref, pts_ref, w0xT_ref, w0fT_ref,
             idx_ref, g_ref, u_ref, V_ref, A_ref):
    b = pl.program_id(0)
    xb = xyz_ref[0]
    xT = xyzT_ref[0]
    inner = jnp.dot(xb, xT)
    quad_row = jnp.sum(xb * xb, axis=1, keepdims=True)
    quad_col = jnp.sum(xT * xT, axis=0, keepdims=True)
    d = (-2.0 * inner + quad_col) + quad_row
    INF = jnp.float32(jnp.inf)
    iota = jax.lax.broadcasted_iota(jnp.int32, (R1, N), 1)
    cols = []
    for t in range(K + 1):
        arg = jnp.argmin(d, axis=1).astype(jnp.int32)[:, None]
        d = jnp.where(iota == arg, INF, d)
        if t > 0:
            cols.append(arg)
    idx_ref[0] = jnp.concatenate(cols, axis=1) + b * N
    u = jnp.dot(xb, w0xT_ref[...])
    u_ref[0] = u
    g_ref[0] = u + jnp.dot(pts_ref[0], w0fT_ref[...])


def _knn_g_u(xyz, points, W0):
    xyz_p = jnp.pad(xyz, ((0, 0), (0, 0), (0, 5)))
    xyzT = jnp.transpose(xyz_p, (0, 2, 1))
    w0xT = jnp.pad(W0[:, :3].T, ((0, 5), (0, 0)))
    w0fT = W0[:, 3:].T
    return pl.pallas_call(
        _k1_body,
        grid=(B, N // R1),
        in_specs=[
            pl.BlockSpec((1, R1, 8), lambda b, r: (b, r, 0)),
            pl.BlockSpec((1, 8, N), lambda b, r: (b, 0, 0)),
            pl.BlockSpec((1, R1, 64), lambda b, r: (b, r, 0)),
            pl.BlockSpec((8, 128), lambda b, r: (0, 0)),
            pl.BlockSpec((64, 128), lambda b, r: (0, 0)),
        ],
        out_specs=[
            pl.BlockSpec((1, R1, K), lambda b, r: (b, r, 0)),
            pl.BlockSpec((1, R1, 128), lambda b, r: (b, r, 0)),
            pl.BlockSpec((1, R1, 128), lambda b, r: (b, r, 0)),
        ],
        out_shape=[
            jax.ShapeDtypeStruct((B, N, K), jnp.int32),
            jax.ShapeDtypeStruct((B, N, 128), jnp.float32),
            jax.ShapeDtypeStruct((B, N, 128), jnp.float32),
        ],
        scratch_shapes=[
            pltpu.VMEM((R1, NCAND), jnp.float32),
            pltpu.VMEM((R1, NCAND), jnp.int32),
        ],
    )(xyz_p, xyzT, points, w0xT, w0fT)


_NW = 32
_CH = 512


def _sc_gather(g_flat, idx_flat):
    per_w = TOT // _NW
    mesh = plsc.VectorSubcoreMesh(core_axis_name="c", subcore_axis_name="s")

    @functools.partial(
        pl.kernel, mesh=mesh,
        out_type=jax.ShapeDtypeStruct((TOT, 128), jnp.float32),
        scratch_types=[
            pltpu.VMEM((_CH,), jnp.int32),
            pltpu.VMEM((_CH, 128), jnp.float32),
            pltpu.SemaphoreType.DMA,
        ],
    )
    def k(g_hbm, idx_hbm, out_hbm, idx_v, rows_v, sem):
        wid = jax.lax.axis_index("s") * 2 + jax.lax.axis_index("c")
        base = wid * per_w

        def body(i, carry):
            off = base + i * _CH
            pltpu.sync_copy(idx_hbm.at[pl.ds(off, _CH)], idx_v)
            pltpu.async_copy(g_hbm.at[idx_v], rows_v, sem).wait()
            pltpu.sync_copy(rows_v, out_hbm.at[pl.ds(off, _CH)])
            return carry

        jax.lax.fori_loop(0, per_w // _CH, body, 0)

    return k(g_flat, idx_flat)


def _k3_body(G_ref, u_ref, s_ref, q_ref):
    step = pl.program_id(0)
    u = u_ref[...]
    u_rep = jnp.broadcast_to(u[:, None, :], (R4 // K, K, 128)).reshape(R4, 128)
    z = G_ref[...] - u_rep
    ps = jnp.sum(z, axis=0, keepdims=True)
    pq = jnp.sum(z * z, axis=0, keepdims=True)

    @pl.when(step == 0)
    def _():
        s_ref[...] = ps
        q_ref[...] = pq

    @pl.when(step != 0)
    def _():
        s_ref[...] += ps
        q_ref[...] += pq


def _z0_stats(G, u_flat):
    return pl.pallas_call(
        _k3_body,
        grid=(TOT // R4,),
        in_specs=[
            pl.BlockSpec((R4, 128), lambda i: (i, 0)),
            pl.BlockSpec((R4 // K, 128), lambda i: (i, 0)),
        ],
        out_specs=[
            pl.BlockSpec((1, 128), lambda i: (0, 0)),
            pl.BlockSpec((1, 128), lambda i: (0, 0)),
        ],
        out_shape=[
            jax.ShapeDtypeStruct((1, 128), jnp.float32),
            jax.ShapeDtypeStruct((1, 128), jnp.float32),
        ],
    )(G, u_flat)


def _k4_body(G_ref, u_ref, a_ref, c_ref, w_ref, z_ref, s_ref, q_ref):
    step = pl.program_id(0)
    u = u_ref[...]
    u_rep = jnp.broadcast_to(u[:, None, :], (R4 // K, K, 128)).reshape(R4, 128)
    z0 = G_ref[...] - u_rep
    h0 = jnp.maximum(z0 * a_ref[...] + c_ref[...], 0.0)
    z1 = jnp.dot(h0.astype(jnp.bfloat16), w_ref[...],
                 preferred_element_type=jnp.float32)
    z_ref[...] = z1.astype(jnp.bfloat16)
    ps = jnp.sum(z1, axis=0, keepdims=True)
    pq = jnp.sum(z1 * z1, axis=0, keepdims=True)

    @pl.when(step == 0)
    def _():
        s_ref[...] = ps
        q_ref[...] = pq

    @pl.when(step != 0)
    def _():
        s_ref[...] += ps
        q_ref[...] += pq


def _layer1(G, u_flat, a0, c0, W1T):
    return pl.pallas_call(
        _k4_body,
        grid=(TOT // R4,),
        in_specs=[
            pl.BlockSpec((R4, 128), lambda i: (i, 0)),
            pl.BlockSpec((R4 // K, 128), lambda i: (i, 0)),
            pl.BlockSpec((1, 128), lambda i: (0, 0)),
            pl.BlockSpec((1, 128), lambda i: (0, 0)),
            pl.BlockSpec((128, 128), lambda i: (0, 0)),
        ],
        out_specs=[
            pl.BlockSpec((R4, 128), lambda i: (i, 0)),
            pl.BlockSpec((1, 128), lambda i: (0, 0)),
            pl.BlockSpec((1, 128), lambda i: (0, 0)),
        ],
        out_shape=[
            jax.ShapeDtypeStruct((TOT, 128), jnp.bfloat16),
            jax.ShapeDtypeStruct((1, 128), jnp.float32),
            jax.ShapeDtypeStruct((1, 128), jnp.float32),
        ],
    )(G, u_flat, a0, c0, W1T)


def _k5_body(z1_ref, a_ref, c_ref, w_ref, m_ref, s_ref, q_ref):
    step = pl.program_id(0)
    z1 = z1_ref[...].astype(jnp.float32)
    h1 = jnp.maximum(z1 * a_ref[...] + c_ref[...], 0.0)
    z2 = jnp.dot(h1.astype(jnp.bfloat16), w_ref[...],
                 preferred_element_type=jnp.float32)
    ps = jnp.sum(z2, axis=0, keepdims=True)
    pq = jnp.sum(z2 * z2, axis=0, keepdims=True)
    m_ref[...] = jnp.max(z2.reshape(R4 // K, K, 256), axis=1)

    @pl.when(step == 0)
    def _():
        s_ref[...] = ps
        q_ref[...] = pq

    @pl.when(step != 0)
    def _():
        s_ref[...] += ps
        q_ref[...] += pq


def _layer2(z1, a1, c1, W2T):
    return pl.pallas_call(
        _k5_body,
        grid=(TOT // R4,),
        in_specs=[
            pl.BlockSpec((R4, 128), lambda i: (i, 0)),
            pl.BlockSpec((1, 128), lambda i: (0, 0)),
            pl.BlockSpec((1, 128), lambda i: (0, 0)),
            pl.BlockSpec((128, 256), lambda i: (0, 0)),
        ],
        out_specs=[
            pl.BlockSpec((R4 // K, 256), lambda i: (i, 0)),
            pl.BlockSpec((1, 256), lambda i: (0, 0)),
            pl.BlockSpec((1, 256), lambda i: (0, 0)),
        ],
        out_shape=[
            jax.ShapeDtypeStruct((B * N, 256), jnp.float32),
            jax.ShapeDtypeStruct((1, 256), jnp.float32),
            jax.ShapeDtypeStruct((1, 256), jnp.float32),
        ],
    )(z1, a1, c1, W2T)


def _k6_body(m_ref, a_ref, c_ref, o_ref):
    o_ref[...] = jnp.maximum(m_ref[...] * a_ref[...] + c_ref[...], 0.0)


def _finalize(M, a2, c2):
    RB = 2048
    return pl.pallas_call(
        _k6_body,
        grid=(B * N // RB,),
        in_specs=[
            pl.BlockSpec((RB, 256), lambda i: (i, 0)),
            pl.BlockSpec((1, 256), lambda i: (0, 0)),
            pl.BlockSpec((1, 256), lambda i: (0, 0)),
        ],
        out_specs=pl.BlockSpec((RB, 256), lambda i: (i, 0)),
        out_shape=jax.ShapeDtypeStruct((B * N, 256), jnp.float32),
    )(M, a2, c2)


def _norm_coeffs(s, q):
    cnt = float(TOT)
    m = s / cnt
    v = q / cnt - m * m
    inv = jax.lax.rsqrt(v + EPS)
    return inv, -m * inv


def kernel(xyz, points, W0, b0, W1, b1, W2, b2):
    idx, g, u = _knn_g_u(xyz, points, W0)
    g_flat = g.reshape(B * N, 128)
    u_flat = u.reshape(B * N, 128)
    G = jnp.take(g_flat, idx.reshape(-1), axis=0)
    s0, q0 = _z0_stats(G, u_flat)
    a0, c0 = _norm_coeffs(s0, q0)
    z1, s1, q1 = _layer1(G, u_flat, a0, c0, W1.T.astype(jnp.bfloat16))
    a1, c1 = _norm_coeffs(s1, q1)
    M, s2, q2 = _layer2(z1, a1, c1, W2.T.astype(jnp.bfloat16))
    a2, c2 = _norm_coeffs(s2, q2)
    out = _finalize(M, a2, c2)
    return out.reshape(B, N, 256)


# speedup vs baseline: 1.7732x; 1.7732x over previous
import functools

import jax
import jax.numpy as jnp
from jax.experimental import pallas as pl
from jax.experimental.pallas import tpu as pltpu
from jax.experimental.pallas import tpu_sc as plsc

B, N, K = 8, 2048, 32
R1 = 512
R4 = 1024
TOT = B * N * K
EPS = 1e-5


CH1 = 128
NCH = N // CH1
J1 = 10
NCAND = NCH * J1


def _k1_body(xyz_ref, xyzT_ref, pts_ref, w0xT_ref, w0fT_ref,
             idx_ref, g_ref, u_ref, V_ref, A_ref):
    b = pl.program_id(0)
    xb = xyz_ref[0]
    xT = xyzT_ref[0]
    inner = jnp.dot(xb, xT)
    quad_row = jnp.sum(xb * xb, axis=1, keepdims=True)
    quad_col = jnp.sum(xT * xT, axis=0, keepdims=True)
    d = (-2.0 * inner + quad_col) + quad_row
    INF = jnp.float32(jnp.inf)
    iota = jax.lax.broadcasted_iota(jnp.int32, (R1, N), 1)
    cols = []
    for t in range(K + 1):
        arg = jnp.argmin(d, axis=1).astype(jnp.int32)[:, None]
        d = jnp.where(iota == arg, INF, d)
        if t > 0:
            cols.append(arg)
    idx_ref[0] = jnp.concatenate(cols, axis=1) + b * N
    u = jnp.dot(xb, w0xT_ref[...])
    u_ref[0] = u
    g_ref[0] = u + jnp.dot(pts_ref[0], w0fT_ref[...])


def _knn_g_u(xyz, points, W0):
    xyz_p = jnp.pad(xyz, ((0, 0), (0, 0), (0, 5)))
    xyzT = jnp.transpose(xyz_p, (0, 2, 1))
    w0xT = jnp.pad(W0[:, :3].T, ((0, 5), (0, 0)))
    w0fT = W0[:, 3:].T
    return pl.pallas_call(
        _k1_body,
        grid=(B, N // R1),
        in_specs=[
            pl.BlockSpec((1, R1, 8), lambda b, r: (b, r, 0)),
            pl.BlockSpec((1, 8, N), lambda b, r: (b, 0, 0)),
            pl.BlockSpec((1, R1, 64), lambda b, r: (b, r, 0)),
            pl.BlockSpec((8, 128), lambda b, r: (0, 0)),
            pl.BlockSpec((64, 128), lambda b, r: (0, 0)),
        ],
        out_specs=[
            pl.BlockSpec((1, R1, K), lambda b, r: (b, r, 0)),
            pl.BlockSpec((1, R1, 128), lambda b, r: (b, r, 0)),
            pl.BlockSpec((1, R1, 128), lambda b, r: (b, r, 0)),
        ],
        out_shape=[
            jax.ShapeDtypeStruct((B, N, K), jnp.int32),
            jax.ShapeDtypeStruct((B, N, 128), jnp.float32),
            jax.ShapeDtypeStruct((B, N, 128), jnp.float32),
        ],
        scratch_shapes=[
            pltpu.VMEM((R1, NCAND), jnp.float32),
            pltpu.VMEM((R1, NCAND), jnp.int32),
        ],
    )(xyz_p, xyzT, points, w0xT, w0fT)


_NW = 32
_CH = 256


def _sc_gather(g_flat, idx_flat):
    per_w = TOT // _NW
    mesh = plsc.VectorSubcoreMesh(core_axis_name="c", subcore_axis_name="s")

    @functools.partial(
        pl.kernel, mesh=mesh,
        out_type=jax.ShapeDtypeStruct((TOT, 128), jnp.float32),
        scratch_types=[
            pltpu.VMEM((_CH,), jnp.int32),
            pltpu.VMEM((_CH,), jnp.int32),
            pltpu.VMEM((_CH, 128), jnp.float32),
            pltpu.VMEM((_CH, 128), jnp.float32),
            pltpu.SemaphoreType.DMA,
            pltpu.SemaphoreType.DMA,
        ],
    )
    def k(g_hbm, idx_hbm, out_hbm, idx_v0, idx_v1, rows_v0, rows_v1,
          sem0, sem1):
        wid = jax.lax.axis_index("s") * 2 + jax.lax.axis_index("c")
        base = wid * per_w

        def body(j, carry):
            off0 = base + (2 * j) * _CH
            off1 = off0 + _CH
            pltpu.sync_copy(idx_hbm.at[pl.ds(off0, _CH)], idx_v0)
            h0 = pltpu.async_copy(g_hbm.at[idx_v0], rows_v0, sem0)
            pltpu.sync_copy(idx_hbm.at[pl.ds(off1, _CH)], idx_v1)
            h1 = pltpu.async_copy(g_hbm.at[idx_v1], rows_v1, sem1)
            h0.wait()
            pltpu.sync_copy(rows_v0, out_hbm.at[pl.ds(off0, _CH)])
            h1.wait()
            pltpu.sync_copy(rows_v1, out_hbm.at[pl.ds(off1, _CH)])
            return carry

        jax.lax.fori_loop(0, per_w // (2 * _CH), body, 0)

    return k(g_flat, idx_flat)


def _k3_body(G_ref, u_ref, s_ref, q_ref):
    step = pl.program_id(0)
    u = u_ref[...]
    u_rep = jnp.broadcast_to(u[:, None, :], (R4 // K, K, 128)).reshape(R4, 128)
    z = G_ref[...] - u_rep
    ps = jnp.sum(z, axis=0, keepdims=True)
    pq = jnp.sum(z * z, axis=0, keepdims=True)

    @pl.when(step == 0)
    def _():
        s_ref[...] = ps
        q_ref[...] = pq

    @pl.when(step != 0)
    def _():
        s_ref[...] += ps
        q_ref[...] += pq


def _z0_stats(G, u_flat):
    return pl.pallas_call(
        _k3_body,
        grid=(TOT // R4,),
        in_specs=[
            pl.BlockSpec((R4, 128), lambda i: (i, 0)),
            pl.BlockSpec((R4 // K, 128), lambda i: (i, 0)),
        ],
        out_specs=[
            pl.BlockSpec((1, 128), lambda i: (0, 0)),
            pl.BlockSpec((1, 128), lambda i: (0, 0)),
        ],
        out_shape=[
            jax.ShapeDtypeStruct((1, 128), jnp.float32),
            jax.ShapeDtypeStruct((1, 128), jnp.float32),
        ],
    )(G, u_flat)


def _k4_body(G_ref, u_ref, a_ref, c_ref, w_ref, z_ref, s_ref, q_ref):
    step = pl.program_id(0)
    u = u_ref[...]
    u_rep = jnp.broadcast_to(u[:, None, :], (R4 // K, K, 128)).reshape(R4, 128)
    z0 = G_ref[...] - u_rep
    h0 = jnp.maximum(z0 * a_ref[...] + c_ref[...], 0.0)
    z1 = jnp.dot(h0.astype(jnp.bfloat16), w_ref[...],
                 preferred_element_type=jnp.float32)
    z_ref[...] = z1.astype(jnp.bfloat16)
    ps = jnp.sum(z1, axis=0, keepdims=True)
    pq = jnp.sum(z1 * z1, axis=0, keepdims=True)

    @pl.when(step == 0)
    def _():
        s_ref[...] = ps
        q_ref[...] = pq

    @pl.when(step != 0)
    def _():
        s_ref[...] += ps
        q_ref[...] += pq


def _layer1(G, u_flat, a0, c0, W1T):
    return pl.pallas_call(
        _k4_body,
        grid=(TOT // R4,),
        in_specs=[
            pl.BlockSpec((R4, 128), lambda i: (i, 0)),
            pl.BlockSpec((R4 // K, 128), lambda i: (i, 0)),
            pl.BlockSpec((1, 128), lambda i: (0, 0)),
            pl.BlockSpec((1, 128), lambda i: (0, 0)),
            pl.BlockSpec((128, 128), lambda i: (0, 0)),
        ],
        out_specs=[
            pl.BlockSpec((R4, 128), lambda i: (i, 0)),
            pl.BlockSpec((1, 128), lambda i: (0, 0)),
            pl.BlockSpec((1, 128), lambda i: (0, 0)),
        ],
        out_shape=[
            jax.ShapeDtypeStruct((TOT, 128), jnp.bfloat16),
            jax.ShapeDtypeStruct((1, 128), jnp.float32),
            jax.ShapeDtypeStruct((1, 128), jnp.float32),
        ],
    )(G, u_flat, a0, c0, W1T)


def _k5_body(z1_ref, a_ref, c_ref, w_ref, m_ref, s_ref, q_ref):
    step = pl.program_id(0)
    z1 = z1_ref[...].astype(jnp.float32)
    h1 = jnp.maximum(z1 * a_ref[...] + c_ref[...], 0.0)
    z2 = jnp.dot(h1.astype(jnp.bfloat16), w_ref[...],
                 preferred_element_type=jnp.float32)
    ps = jnp.sum(z2, axis=0, keepdims=True)
    pq = jnp.sum(z2 * z2, axis=0, keepdims=True)
    m_ref[...] = jnp.max(z2.reshape(R4 // K, K, 256), axis=1)

    @pl.when(step == 0)
    def _():
        s_ref[...] = ps
        q_ref[...] = pq

    @pl.when(step != 0)
    def _():
        s_ref[...] += ps
        q_ref[...] += pq


def _layer2(z1, a1, c1, W2T):
    return pl.pallas_call(
        _k5_body,
        grid=(TOT // R4,),
        in_specs=[
            pl.BlockSpec((R4, 128), lambda i: (i, 0)),
            pl.BlockSpec((1, 128), lambda i: (0, 0)),
            pl.BlockSpec((1, 128), lambda i: (0, 0)),
            pl.BlockSpec((128, 256), lambda i: (0, 0)),
        ],
        out_specs=[
            pl.BlockSpec((R4 // K, 256), lambda i: (i, 0)),
            pl.BlockSpec((1, 256), lambda i: (0, 0)),
            pl.BlockSpec((1, 256), lambda i: (0, 0)),
        ],
        out_shape=[
            jax.ShapeDtypeStruct((B * N, 256), jnp.float32),
            jax.ShapeDtypeStruct((1, 256), jnp.float32),
            jax.ShapeDtypeStruct((1, 256), jnp.float32),
        ],
    )(z1, a1, c1, W2T)


def _k6_body(m_ref, a_ref, c_ref, o_ref):
    o_ref[...] = jnp.maximum(m_ref[...] * a_ref[...] + c_ref[...], 0.0)


def _finalize(M, a2, c2):
    RB = 2048
    return pl.pallas_call(
        _k6_body,
        grid=(B * N // RB,),
        in_specs=[
            pl.BlockSpec((RB, 256), lambda i: (i, 0)),
            pl.BlockSpec((1, 256), lambda i: (0, 0)),
            pl.BlockSpec((1, 256), lambda i: (0, 0)),
        ],
        out_specs=pl.BlockSpec((RB, 256), lambda i: (i, 0)),
        out_shape=jax.ShapeDtypeStruct((B * N, 256), jnp.float32),
    )(M, a2, c2)


def _norm_coeffs(s, q):
    cnt = float(TOT)
    m = s / cnt
    v = q / cnt - m * m
    inv = jax.lax.rsqrt(v + EPS)
    return inv, -m * inv


def kernel(xyz, points, W0, b0, W1, b1, W2, b2):
    idx, g, u = _knn_g_u(xyz, points, W0)
    g_flat = g.reshape(B * N, 128)
    u_flat = u.reshape(B * N, 128)
    G = _sc_gather(g_flat, idx.reshape(-1))
    s0, q0 = _z0_stats(G, u_flat)
    a0, c0 = _norm_coeffs(s0, q0)
    z1, s1, q1 = _layer1(G, u_flat, a0, c0, W1.T.astype(jnp.bfloat16))
    a1, c1 = _norm_coeffs(s1, q1)
    M, s2, q2 = _layer2(z1, a1, c1, W2.T.astype(jnp.bfloat16))
    a2, c2 = _norm_coeffs(s2, q2)
    out = _finalize(M, a2, c2)
    return out.reshape(B, N, 256)
